# baseline (speedup 1.0000x reference)
import functools

import numpy as np
import jax
import jax.numpy as jnp
from jax import lax
from jax.experimental import pallas as pl
from jax.experimental.pallas import tpu as pltpu
from jax.experimental.pallas import tpu_sc as plsc

_D = 64
_S = 200
_CH = 100


def _pos_encoding_np(max_pos: int, d: int) -> np.ndarray:
    pos = np.arange(max_pos)[:, None].astype(np.float32)
    i = np.arange(d)[None, :]
    rates = 1.0 / np.power(10000.0, 2 * (i // 2) / np.float32(d))
    ang = pos * rates
    ang[:, 0::2] = np.sin(ang[:, 0::2])
    ang[:, 1::2] = np.cos(ang[:, 1::2])
    return ang.astype(np.float32)


_PE = _pos_encoding_np(256, _D)[:_S]


def kernel(x_in, table):
    B, S = x_in.shape
    D = table.shape[1]
    N = B * S
    info = plsc.get_sparse_core_info()
    NC, NS = info.num_cores, info.num_subcores
    NW = NC * NS
    n_per_w = N // NW
    seq_per_w = n_per_w // S
    chunks = n_per_w // _CH
    SP = S // 2

    x3 = x_in.astype(jnp.int32).reshape(NW, chunks, _CH)
    pe = jnp.asarray(_PE)

    mesh = plsc.VectorSubcoreMesh(core_axis_name="c", subcore_axis_name="s")

    @functools.partial(
        pl.kernel,
        mesh=mesh,
        out_type=jax.ShapeDtypeStruct((N // 2, 2 * D), jnp.float32),
        scratch_types=[
            pltpu.VMEM((chunks, _CH), jnp.int32),
            pltpu.VMEM((S, D), jnp.float32),
            pltpu.VMEM((2, S, D), jnp.float32),
            pltpu.VMEM((2, SP, 2 * D), jnp.float32),
        ] + [pltpu.SemaphoreType.DMA] * 4,
        compiler_params=pltpu.CompilerParams(use_tc_tiling_on_sc=False),
    )
    def run(x_hbm, table_hbm, pe_hbm, out_hbm, idx_v, pe_v, gbuf, obuf, *sems):
        gsem = sems[:2]
        osem = sems[2:]
        wid = lax.axis_index("s") * NC + lax.axis_index("c")
        pltpu.sync_copy(pe_hbm, pe_v)
        pltpu.sync_copy(x_hbm.at[wid], idx_v)
        row0 = wid * seq_per_w

        def issue_gather(j, b):
            c0 = 2 * j
            pltpu.async_copy(
                table_hbm.at[idx_v.at[c0]], gbuf.at[b, pl.ds(0, _CH)],
                gsem[b])
            pltpu.async_copy(
                table_hbm.at[idx_v.at[c0 + 1]], gbuf.at[b, pl.ds(_CH, _CH)],
                gsem[b])

        def drain_g(sem, b):
            pltpu.make_async_copy(
                table_hbm.at[pl.ds(0, S)], gbuf.at[b], sem).wait()

        def drain_o(sem, b):
            pltpu.make_async_copy(
                out_hbm.at[pl.ds(0, SP)], obuf.at[b], sem).wait()

        issue_gather(0, 0)

        @pl.loop(0, seq_per_w, step=2)
        def _(jj):
            for b in range(2):
                j = jj + b

                drain_g(gsem[b], b)

                @pl.when(j + 1 < seq_per_w)
                def _():
                    issue_gather(j + 1, 1 - b)

                @pl.when(j >= 2)
                def _():
                    drain_o(osem[b], b)

                @pl.loop(0, SP, unroll=2)
                def _(q):
                    for h in range(2):
                        for k in range(D // 16):
                            src = pl.ds(k * 16, 16)
                            dst = pl.ds(h * D + k * 16, 16)
                            obuf[b, q, dst] = (
                                gbuf[b, 2 * q + h, src] + pe_v[2 * q + h, src])

                pltpu.async_copy(
                    obuf.at[b], out_hbm.at[pl.ds((row0 + j) * SP, SP)],
                    osem[b])

        for b in range(2):
            drain_o(osem[b], b)

    out = run(x3, table, pe)
    return out.reshape(B, S, D)


# speedup vs baseline: 1.4549x; 1.4549x over previous
import functools

import numpy as np
import jax
import jax.numpy as jnp
from jax import lax
from jax.experimental import pallas as pl
from jax.experimental.pallas import tpu as pltpu
from jax.experimental.pallas import tpu_sc as plsc

_D = 64
_S = 200
_CH = 100


def _pos_encoding_np(max_pos: int, d: int) -> np.ndarray:
    pos = np.arange(max_pos)[:, None].astype(np.float32)
    i = np.arange(d)[None, :]
    rates = 1.0 / np.power(10000.0, 2 * (i // 2) / np.float32(d))
    ang = pos * rates
    ang[:, 0::2] = np.sin(ang[:, 0::2])
    ang[:, 1::2] = np.cos(ang[:, 1::2])
    return ang.astype(np.float32)


_PE = _pos_encoding_np(256, _D)[:_S]


def kernel(x_in, table):
    B, S = x_in.shape
    D = table.shape[1]
    N = B * S
    info = plsc.get_sparse_core_info()
    NC, NS = info.num_cores, info.num_subcores
    NW = NC * NS
    n_per_w = N // NW
    seq_per_w = n_per_w // S
    chunks = n_per_w // _CH
    SP = S // 2

    x3 = x_in.astype(jnp.int32).reshape(NW, chunks, _CH)
    pe = jnp.asarray(_PE)

    mesh = plsc.VectorSubcoreMesh(core_axis_name="c", subcore_axis_name="s")

    @functools.partial(
        pl.kernel,
        mesh=mesh,
        out_type=jax.ShapeDtypeStruct((N // 2, 2 * D), jnp.float32),
        scratch_types=[
            pltpu.VMEM((chunks, _CH), jnp.int32),
            pltpu.VMEM((S, D), jnp.float32),
            pltpu.VMEM((2, S, D), jnp.float32),
            pltpu.VMEM((2, SP, 2 * D), jnp.float32),
        ] + [pltpu.SemaphoreType.DMA] * 4,
        compiler_params=pltpu.CompilerParams(use_tc_tiling_on_sc=False),
    )
    def run(x_hbm, table_hbm, pe_hbm, out_hbm, idx_v, pe_v, gbuf, obuf, *sems):
        gsem = sems[:2]
        osem = sems[2:]
        wid = lax.axis_index("s") * NC + lax.axis_index("c")
        pltpu.sync_copy(pe_hbm, pe_v)
        pltpu.sync_copy(x_hbm.at[wid], idx_v)
        row0 = wid * seq_per_w

        def issue_gather(j, b):
            c0 = 2 * j
            pltpu.async_copy(
                table_hbm.at[idx_v.at[c0]], gbuf.at[b, pl.ds(0, _CH)],
                gsem[b])
            pltpu.async_copy(
                table_hbm.at[idx_v.at[c0 + 1]], gbuf.at[b, pl.ds(_CH, _CH)],
                gsem[b])

        def drain_g(sem, b):
            pltpu.make_async_copy(
                table_hbm.at[pl.ds(0, S)], gbuf.at[b], sem).wait()

        def drain_o(sem, b):
            pltpu.make_async_copy(
                out_hbm.at[pl.ds(0, SP)], obuf.at[b], sem).wait()

        issue_gather(0, 0)

        @pl.loop(0, seq_per_w, step=2)
        def _(jj):
            for b in range(2):
                j = jj + b

                drain_g(gsem[b], b)

                @pl.when(j + 1 < seq_per_w)
                def _():
                    issue_gather(j + 1, 1 - b)

                @pl.when(j >= 2)
                def _():
                    drain_o(osem[b], b)

                @plsc.parallel_loop(0, SP, unroll=4)
                def _(q):
                    for h in range(2):
                        for k in range(D // 16):
                            src = pl.ds(k * 16, 16)
                            dst = pl.ds(h * D + k * 16, 16)
                            obuf[b, q, dst] = (
                                gbuf[b, 2 * q + h, src] + pe_v[2 * q + h, src])

                pltpu.async_copy(
                    obuf.at[b], out_hbm.at[pl.ds((row0 + j) * SP, SP)],
                    osem[b])

        for b in range(2):
            drain_o(osem[b], b)

    out = run(x3, table, pe)
    return out.reshape(B, S, D)
